# Initial kernel scaffold; baseline (speedup 1.0000x reference)
import jax
import jax.numpy as jnp
from jax.experimental import pallas as pl


def kernel(x, edge_index, W1, b1, W2, b2):
    raise NotImplementedError("write your pallas kernel here")


# speedup vs baseline: 29.5705x; 29.5705x over previous
import functools

import jax
import jax.numpy as jnp
from jax import lax
from jax.experimental import pallas as pl
from jax.experimental.pallas import tpu as pltpu
from jax.experimental.pallas import tpu_sc as plsc

N_NODES = 10000
N_EDGES = 320000
NC = 2
NS = 16
NW = NC * NS
CHUNK = 125
EDGES_PER_TILE = N_EDGES // NW
CHUNKS_PER_TILE = EDGES_PER_TILE // CHUNK
ROWS_PER_TILE = N_NODES // NS
OUT_SLICE = 624
OUT_TAIL = N_NODES - NS * OUT_SLICE
DEG_W = 16

ROW_BLK = 1000


def _sc_mesh():
    return plsc.VectorSubcoreMesh(core_axis_name="c", subcore_axis_name="s")


def _sc_deg(dst2d):
    @functools.partial(
        pl.kernel,
        out_type=jax.ShapeDtypeStruct((NC, N_NODES, DEG_W), jnp.float32),
        mesh=_sc_mesh(),
        compiler_params=pltpu.CompilerParams(use_tc_tiling_on_sc=False),
        scratch_types=[
            pltpu.VMEM((CHUNKS_PER_TILE, CHUNK), jnp.int32),
            pltpu.VMEM((CHUNK, DEG_W), jnp.float32),
            pltpu.VMEM_SHARED((N_NODES, DEG_W), jnp.float32),
        ],
    )
    def k(dst_hbm, out_hbm, dst_v, ones_v, acc):
        c = lax.axis_index("c")
        s = lax.axis_index("s")
        wid = c * NS + s
        base = s * ROWS_PER_TILE

        def zrow(i, _):
            ones_v[i, :] = jnp.zeros((DEG_W,), jnp.float32)
            return 0

        lax.fori_loop(0, CHUNK, zrow, 0)
        for q in range(ROWS_PER_TILE // CHUNK):
            pltpu.sync_copy(ones_v, acc.at[pl.ds(base + q * CHUNK, CHUNK)])

        pltpu.sync_copy(
            dst_hbm.at[pl.ds(wid * CHUNKS_PER_TILE, CHUNKS_PER_TILE)], dst_v)

        def orow(i, _):
            ones_v[i, :] = jnp.ones((DEG_W,), jnp.float32)
            return 0

        lax.fori_loop(0, CHUNK, orow, 0)
        plsc.subcore_barrier()

        def body(j, _):
            pltpu.sync_copy(ones_v, acc.at[dst_v.at[j]], add=True)
            return 0

        lax.fori_loop(0, CHUNKS_PER_TILE, body, 0)
        plsc.subcore_barrier()
        off = pl.multiple_of(s * OUT_SLICE, 8)
        pltpu.sync_copy(acc.at[pl.ds(off, OUT_SLICE)],
                        out_hbm.at[c, pl.ds(off, OUT_SLICE)])

        @pl.when(s == 0)
        def _tail():
            pltpu.sync_copy(acc.at[pl.ds(NS * OUT_SLICE, OUT_TAIL)],
                            out_hbm.at[c, pl.ds(NS * OUT_SLICE, OUT_TAIL)])

    return k(dst2d)


def _sc_agg(y, src2d, dst2d, feat):
    @functools.partial(
        pl.kernel,
        out_type=jax.ShapeDtypeStruct((NC, N_NODES, feat), jnp.float32),
        mesh=_sc_mesh(),
        compiler_params=pltpu.CompilerParams(use_tc_tiling_on_sc=False),
        scratch_types=[
            pltpu.VMEM((CHUNKS_PER_TILE, CHUNK), jnp.int32),
            pltpu.VMEM((CHUNKS_PER_TILE, CHUNK), jnp.int32),
            pltpu.VMEM((CHUNK, feat), jnp.float32),
            pltpu.VMEM_SHARED((N_NODES, feat), jnp.float32),
            pltpu.SemaphoreType.DMA,
        ],
    )
    def k(y_hbm, src_hbm, dst_hbm, out_hbm, src_v, dst_v, rows_v, acc, sem):
        c = lax.axis_index("c")
        s = lax.axis_index("s")
        wid = c * NS + s
        base = s * ROWS_PER_TILE

        def zrow(i, _):
            for t in range(feat // 16):
                rows_v[i, pl.ds(t * 16, 16)] = jnp.zeros((16,), jnp.float32)
            return 0

        lax.fori_loop(0, CHUNK, zrow, 0)
        for q in range(ROWS_PER_TILE // CHUNK):
            pltpu.sync_copy(rows_v, acc.at[pl.ds(base + q * CHUNK, CHUNK)])

        pltpu.sync_copy(
            src_hbm.at[pl.ds(wid * CHUNKS_PER_TILE, CHUNKS_PER_TILE)], src_v)
        pltpu.sync_copy(
            dst_hbm.at[pl.ds(wid * CHUNKS_PER_TILE, CHUNKS_PER_TILE)], dst_v)
        plsc.subcore_barrier()

        def body(j, _):
            pltpu.async_copy(y_hbm.at[src_v.at[j]], rows_v, sem).wait()
            pltpu.sync_copy(rows_v, acc.at[dst_v.at[j]], add=True)
            return 0

        lax.fori_loop(0, CHUNKS_PER_TILE, body, 0)
        plsc.subcore_barrier()
        off = pl.multiple_of(s * OUT_SLICE, 8)
        pltpu.sync_copy(acc.at[pl.ds(off, OUT_SLICE)],
                        out_hbm.at[c, pl.ds(off, OUT_SLICE)])

        @pl.when(s == 0)
        def _tail():
            pltpu.sync_copy(acc.at[pl.ds(NS * OUT_SLICE, OUT_TAIL)],
                            out_hbm.at[c, pl.ds(NS * OUT_SLICE, OUT_TAIL)])

    return k(y, src2d, dst2d)


def _tc_stage1(x, W1, degp):
    n, d = x.shape
    h = W1.shape[1]

    def body(x_ref, w_ref, dp_ref, xw_ref, y_ref, dis_ref):
        dp = dp_ref[...]
        deg = dp[0, :, 0:1] + dp[1, :, 0:1] + 1.0
        dis = lax.rsqrt(deg)
        xw = jnp.dot(x_ref[...], w_ref[...],
                     preferred_element_type=jnp.float32)
        xw_ref[...] = xw
        y_ref[...] = xw * dis
        dis_ref[...] = dis

    return pl.pallas_call(
        body,
        grid=(n // ROW_BLK,),
        in_specs=[
            pl.BlockSpec((ROW_BLK, d), lambda i: (i, 0)),
            pl.BlockSpec((d, h), lambda i: (0, 0)),
            pl.BlockSpec((NC, ROW_BLK, DEG_W), lambda i: (0, i, 0)),
        ],
        out_specs=[
            pl.BlockSpec((ROW_BLK, h), lambda i: (i, 0)),
            pl.BlockSpec((ROW_BLK, h), lambda i: (i, 0)),
            pl.BlockSpec((ROW_BLK, 1), lambda i: (i, 0)),
        ],
        out_shape=[
            jax.ShapeDtypeStruct((n, h), jnp.float32),
            jax.ShapeDtypeStruct((n, h), jnp.float32),
            jax.ShapeDtypeStruct((n, 1), jnp.float32),
        ],
    )(x, W1, degp)


def _tc_stage2(aggp, xw1, dis, b1r, W2):
    n, hdim = xw1.shape
    edim = W2.shape[1]

    def body(a_ref, xw_ref, dis_ref, b_ref, w_ref, xw2_ref, y2_ref):
        a = a_ref[...]
        dis = dis_ref[...]
        hid = jnp.maximum(
            dis * (a[0] + a[1]) + dis * dis * xw_ref[...] + b_ref[...], 0.0)
        xw2 = jnp.dot(hid, w_ref[...], preferred_element_type=jnp.float32)
        xw2_ref[...] = xw2
        y2_ref[...] = xw2 * dis

    return pl.pallas_call(
        body,
        grid=(n // ROW_BLK,),
        in_specs=[
            pl.BlockSpec((NC, ROW_BLK, hdim), lambda i: (0, i, 0)),
            pl.BlockSpec((ROW_BLK, hdim), lambda i: (i, 0)),
            pl.BlockSpec((ROW_BLK, 1), lambda i: (i, 0)),
            pl.BlockSpec((1, hdim), lambda i: (0, 0)),
            pl.BlockSpec((hdim, edim), lambda i: (0, 0)),
        ],
        out_specs=[
            pl.BlockSpec((ROW_BLK, edim), lambda i: (i, 0)),
            pl.BlockSpec((ROW_BLK, edim), lambda i: (i, 0)),
        ],
        out_shape=[
            jax.ShapeDtypeStruct((n, edim), jnp.float32),
            jax.ShapeDtypeStruct((n, edim), jnp.float32),
        ],
    )(aggp, xw1, dis, b1r, W2)


def _tc_stage3(aggp, xw2, dis, b2r):
    n, edim = xw2.shape

    def body(a_ref, xw_ref, dis_ref, b_ref, out_ref):
        a = a_ref[...]
        dis = dis_ref[...]
        out_ref[...] = (dis * (a[0] + a[1])
                        + dis * dis * xw_ref[...] + b_ref[...])

    return pl.pallas_call(
        body,
        grid=(n // ROW_BLK,),
        in_specs=[
            pl.BlockSpec((NC, ROW_BLK, edim), lambda i: (0, i, 0)),
            pl.BlockSpec((ROW_BLK, edim), lambda i: (i, 0)),
            pl.BlockSpec((ROW_BLK, 1), lambda i: (i, 0)),
            pl.BlockSpec((1, edim), lambda i: (0, 0)),
        ],
        out_specs=pl.BlockSpec((ROW_BLK, edim), lambda i: (i, 0)),
        out_shape=jax.ShapeDtypeStruct((n, edim), jnp.float32),
    )(aggp, xw2, dis, b2r)


def kernel(x, edge_index, W1, b1, W2, b2):
    ei = edge_index.astype(jnp.int32)
    src2d = ei[0].reshape(NW * CHUNKS_PER_TILE, CHUNK)
    dst2d = ei[1].reshape(NW * CHUNKS_PER_TILE, CHUNK)

    degp = _sc_deg(dst2d)
    xw1, y1, dis = _tc_stage1(x, W1, degp)
    agg1p = _sc_agg(y1, src2d, dst2d, W1.shape[1])
    xw2, y2 = _tc_stage2(agg1p, xw1, dis, b1.reshape(1, -1), W2)
    agg2p = _sc_agg(y2, src2d, dst2d, W2.shape[1])
    return _tc_stage3(agg2p, xw2, dis, b2.reshape(1, -1))


# speedup vs baseline: 37.4472x; 1.2664x over previous
import functools

import jax
import jax.numpy as jnp
from jax import lax
from jax.experimental import pallas as pl
from jax.experimental.pallas import tpu as pltpu
from jax.experimental.pallas import tpu_sc as plsc

N_NODES = 10000
N_EDGES = 320000
NC = 2
NS = 16
NW = NC * NS
CHUNK = 125
EDGES_PER_TILE = N_EDGES // NW
CHUNKS_PER_TILE = EDGES_PER_TILE // CHUNK
ROWS_PER_TILE = N_NODES // NS
OUT_SLICE = 624
OUT_TAIL = N_NODES - NS * OUT_SLICE
DEG_W = 8
DEG_GROUP = 16
PAIRS = CHUNKS_PER_TILE // 2

ROW_BLK = 1000


def _sc_mesh():
    return plsc.VectorSubcoreMesh(core_axis_name="c", subcore_axis_name="s")


def _sc_deg(dst2d):
    @functools.partial(
        pl.kernel,
        out_type=jax.ShapeDtypeStruct((NC, N_NODES, DEG_W), jnp.float32),
        mesh=_sc_mesh(),
        compiler_params=pltpu.CompilerParams(use_tc_tiling_on_sc=False),
        scratch_types=[
            pltpu.VMEM((CHUNKS_PER_TILE, CHUNK), jnp.int32),
            pltpu.VMEM((CHUNK, DEG_W), jnp.float32),
            pltpu.VMEM_SHARED((N_NODES, DEG_W), jnp.float32),
            pltpu.SemaphoreType.DMA,
        ],
    )
    def k(dst_hbm, out_hbm, dst_v, ones_v, acc, sem):
        c = lax.axis_index("c")
        s = lax.axis_index("s")
        wid = c * NS + s
        base = s * ROWS_PER_TILE

        def zrow(i, _):
            ones_v[i, :] = jnp.zeros((DEG_W,), jnp.float32)
            return 0

        lax.fori_loop(0, CHUNK, zrow, 0)
        for q in range(ROWS_PER_TILE // CHUNK):
            pltpu.sync_copy(ones_v, acc.at[pl.ds(base + q * CHUNK, CHUNK)])

        pltpu.sync_copy(
            dst_hbm.at[pl.ds(wid * CHUNKS_PER_TILE, CHUNKS_PER_TILE)], dst_v)

        def orow(i, _):
            ones_v[i, :] = jnp.ones((DEG_W,), jnp.float32)
            return 0

        lax.fori_loop(0, CHUNK, orow, 0)
        plsc.subcore_barrier()

        def body(g, _):
            descs = [
                pltpu.async_copy(
                    ones_v, acc.at[dst_v.at[g * DEG_GROUP + u]], sem, add=True)
                for u in range(DEG_GROUP)
            ]
            for desc in descs:
                desc.wait()
            return 0

        lax.fori_loop(0, CHUNKS_PER_TILE // DEG_GROUP, body, 0)
        plsc.subcore_barrier()
        off = pl.multiple_of(s * OUT_SLICE, 8)
        pltpu.sync_copy(acc.at[pl.ds(off, OUT_SLICE)],
                        out_hbm.at[c, pl.ds(off, OUT_SLICE)])

        @pl.when(s == 0)
        def _tail():
            pltpu.sync_copy(acc.at[pl.ds(NS * OUT_SLICE, OUT_TAIL)],
                            out_hbm.at[c, pl.ds(NS * OUT_SLICE, OUT_TAIL)])

    return k(dst2d)


def _sc_agg(y, src2d, dst2d, feat):
    @functools.partial(
        pl.kernel,
        out_type=jax.ShapeDtypeStruct((NC, N_NODES, feat), jnp.float32),
        mesh=_sc_mesh(),
        compiler_params=pltpu.CompilerParams(use_tc_tiling_on_sc=False),
        scratch_types=[
            pltpu.VMEM((CHUNKS_PER_TILE, CHUNK), jnp.int32),
            pltpu.VMEM((CHUNKS_PER_TILE, CHUNK), jnp.int32),
            pltpu.VMEM((CHUNK, feat), jnp.float32),
            pltpu.VMEM((CHUNK, feat), jnp.float32),
            pltpu.VMEM_SHARED((N_NODES, feat), jnp.float32),
            pltpu.SemaphoreType.DMA,
            pltpu.SemaphoreType.DMA,
            pltpu.SemaphoreType.DMA,
            pltpu.SemaphoreType.DMA,
        ],
    )
    def k(y_hbm, src_hbm, dst_hbm, out_hbm,
          src_v, dst_v, rows0, rows1, acc, gsem0, gsem1, ssem0, ssem1):
        c = lax.axis_index("c")
        s = lax.axis_index("s")
        wid = c * NS + s
        base = s * ROWS_PER_TILE

        def zrow(i, _):
            for t in range(feat // 16):
                rows0[i, pl.ds(t * 16, 16)] = jnp.zeros((16,), jnp.float32)
            return 0

        lax.fori_loop(0, CHUNK, zrow, 0)
        for q in range(ROWS_PER_TILE // CHUNK):
            pltpu.sync_copy(rows0, acc.at[pl.ds(base + q * CHUNK, CHUNK)])

        pltpu.sync_copy(
            src_hbm.at[pl.ds(wid * CHUNKS_PER_TILE, CHUNKS_PER_TILE)], src_v)
        pltpu.sync_copy(
            dst_hbm.at[pl.ds(wid * CHUNKS_PER_TILE, CHUNKS_PER_TILE)], dst_v)
        plsc.subcore_barrier()

        pltpu.async_copy(y_hbm.at[src_v.at[0]], rows0, gsem0)
        pltpu.async_copy(y_hbm.at[src_v.at[1]], rows1, gsem1)

        def body(p, _):
            j0 = 2 * p
            j1 = j0 + 1
            pltpu.make_async_copy(y_hbm.at[src_v.at[j0]], rows0, gsem0).wait()
            s0 = pltpu.async_copy(rows0, acc.at[dst_v.at[j0]], ssem0, add=True)
            pltpu.make_async_copy(y_hbm.at[src_v.at[j1]], rows1, gsem1).wait()
            s1 = pltpu.async_copy(rows1, acc.at[dst_v.at[j1]], ssem1, add=True)
            s0.wait()
            pltpu.async_copy(y_hbm.at[src_v.at[j0 + 2]], rows0, gsem0)
            s1.wait()
            pltpu.async_copy(y_hbm.at[src_v.at[j1 + 2]], rows1, gsem1)
            return 0

        lax.fori_loop(0, PAIRS - 1, body, 0)
        jl0 = 2 * (PAIRS - 1)
        jl1 = jl0 + 1
        pltpu.make_async_copy(y_hbm.at[src_v.at[jl0]], rows0, gsem0).wait()
        sl0 = pltpu.async_copy(rows0, acc.at[dst_v.at[jl0]], ssem0, add=True)
        pltpu.make_async_copy(y_hbm.at[src_v.at[jl1]], rows1, gsem1).wait()
        sl1 = pltpu.async_copy(rows1, acc.at[dst_v.at[jl1]], ssem1, add=True)
        sl0.wait()
        sl1.wait()
        plsc.subcore_barrier()
        off = pl.multiple_of(s * OUT_SLICE, 8)
        pltpu.sync_copy(acc.at[pl.ds(off, OUT_SLICE)],
                        out_hbm.at[c, pl.ds(off, OUT_SLICE)])

        @pl.when(s == 0)
        def _tail():
            pltpu.sync_copy(acc.at[pl.ds(NS * OUT_SLICE, OUT_TAIL)],
                            out_hbm.at[c, pl.ds(NS * OUT_SLICE, OUT_TAIL)])

    return k(y, src2d, dst2d)


def _tc_stage1(x, W1, degp):
    n, d = x.shape
    h = W1.shape[1]

    def body(x_ref, w_ref, dp_ref, xw_ref, y_ref, dis_ref):
        dp = dp_ref[...]
        deg = dp[0, :, 0:1] + dp[1, :, 0:1] + 1.0
        dis = lax.rsqrt(deg)
        xw = jnp.dot(x_ref[...], w_ref[...],
                     preferred_element_type=jnp.float32)
        xw_ref[...] = xw
        y_ref[...] = xw * dis
        dis_ref[...] = dis

    return pl.pallas_call(
        body,
        grid=(n // ROW_BLK,),
        in_specs=[
            pl.BlockSpec((ROW_BLK, d), lambda i: (i, 0)),
            pl.BlockSpec((d, h), lambda i: (0, 0)),
            pl.BlockSpec((NC, ROW_BLK, DEG_W), lambda i: (0, i, 0)),
        ],
        out_specs=[
            pl.BlockSpec((ROW_BLK, h), lambda i: (i, 0)),
            pl.BlockSpec((ROW_BLK, h), lambda i: (i, 0)),
            pl.BlockSpec((ROW_BLK, 1), lambda i: (i, 0)),
        ],
        out_shape=[
            jax.ShapeDtypeStruct((n, h), jnp.float32),
            jax.ShapeDtypeStruct((n, h), jnp.float32),
            jax.ShapeDtypeStruct((n, 1), jnp.float32),
        ],
    )(x, W1, degp)


def _tc_stage2(aggp, xw1, dis, b1r, W2):
    n, hdim = xw1.shape
    edim = W2.shape[1]

    def body(a_ref, xw_ref, dis_ref, b_ref, w_ref, xw2_ref, y2_ref):
        a = a_ref[...]
        dis = dis_ref[...]
        hid = jnp.maximum(
            dis * (a[0] + a[1]) + dis * dis * xw_ref[...] + b_ref[...], 0.0)
        xw2 = jnp.dot(hid, w_ref[...], preferred_element_type=jnp.float32)
        xw2_ref[...] = xw2
        y2_ref[...] = xw2 * dis

    return pl.pallas_call(
        body,
        grid=(n // ROW_BLK,),
        in_specs=[
            pl.BlockSpec((NC, ROW_BLK, hdim), lambda i: (0, i, 0)),
            pl.BlockSpec((ROW_BLK, hdim), lambda i: (i, 0)),
            pl.BlockSpec((ROW_BLK, 1), lambda i: (i, 0)),
            pl.BlockSpec((1, hdim), lambda i: (0, 0)),
            pl.BlockSpec((hdim, edim), lambda i: (0, 0)),
        ],
        out_specs=[
            pl.BlockSpec((ROW_BLK, edim), lambda i: (i, 0)),
            pl.BlockSpec((ROW_BLK, edim), lambda i: (i, 0)),
        ],
        out_shape=[
            jax.ShapeDtypeStruct((n, edim), jnp.float32),
            jax.ShapeDtypeStruct((n, edim), jnp.float32),
        ],
    )(aggp, xw1, dis, b1r, W2)


def _tc_stage3(aggp, xw2, dis, b2r):
    n, edim = xw2.shape

    def body(a_ref, xw_ref, dis_ref, b_ref, out_ref):
        a = a_ref[...]
        dis = dis_ref[...]
        out_ref[...] = (dis * (a[0] + a[1])
                        + dis * dis * xw_ref[...] + b_ref[...])

    return pl.pallas_call(
        body,
        grid=(n // ROW_BLK,),
        in_specs=[
            pl.BlockSpec((NC, ROW_BLK, edim), lambda i: (0, i, 0)),
            pl.BlockSpec((ROW_BLK, edim), lambda i: (i, 0)),
            pl.BlockSpec((ROW_BLK, 1), lambda i: (i, 0)),
            pl.BlockSpec((1, edim), lambda i: (0, 0)),
        ],
        out_specs=pl.BlockSpec((ROW_BLK, edim), lambda i: (i, 0)),
        out_shape=jax.ShapeDtypeStruct((n, edim), jnp.float32),
    )(aggp, xw2, dis, b2r)


def kernel(x, edge_index, W1, b1, W2, b2):
    ei = edge_index.astype(jnp.int32)
    src2d = ei[0].reshape(NW * CHUNKS_PER_TILE, CHUNK)
    dst2d = ei[1].reshape(NW * CHUNKS_PER_TILE, CHUNK)

    degp = _sc_deg(dst2d)
    xw1, y1, dis = _tc_stage1(x, W1, degp)
    agg1p = _sc_agg(y1, src2d, dst2d, W1.shape[1])
    xw2, y2 = _tc_stage2(agg1p, xw1, dis, b1.reshape(1, -1), W2)
    agg2p = _sc_agg(y2, src2d, dst2d, W2.shape[1])
    return _tc_stage3(agg2p, xw2, dis, b2.reshape(1, -1))
